# Initial kernel scaffold; baseline (speedup 1.0000x reference)
import jax
import jax.numpy as jnp
from jax.experimental import pallas as pl


def kernel(h_bond, bond_index, h_node, fL_Wb, fL_Wn, fL_W1, fL_b1, fL_W2, fL_b2, fR_Wb, fR_Wn, fR_W1, fR_b1, fR_W2, fR_b2, Wnl, bnl, Wnr, bnr, Ws, bs, ln_g, ln_b, Wo, bo):
    raise NotImplementedError("write your pallas kernel here")


# speedup vs baseline: 2.0520x; 2.0520x over previous
import functools

import jax
import jax.numpy as jnp
from jax import lax
from jax.experimental import pallas as pl
from jax.experimental.pallas import tpu as pltpu
from jax.experimental.pallas import tpu_sc as plsc

N = 10000
E = 320000
BD = 128
ND = 128
ID = 256

NC = 2
NS = 16
NW = NC * NS

CHUNK = 128
NCHUNK = E // CHUNK
ROWS_A = 624
ROWS_LAST = N - ROWS_A * (NS - 1)

_mesh = plsc.VectorSubcoreMesh(core_axis_name="c", subcore_axis_name="s")


def _gather_pair(tabA, idxA, tabB, idxB):
    niter = (NCHUNK + NW - 1) // NW

    @functools.partial(
        pl.kernel,
        out_type=(
            jax.ShapeDtypeStruct((E, BD), jnp.float32),
            jax.ShapeDtypeStruct((E, BD), jnp.float32),
        ),
        mesh=_mesh,
        scratch_types=(
            pltpu.VMEM((CHUNK,), jnp.int32),
            pltpu.VMEM((CHUNK, BD), jnp.float32),
            pltpu.VMEM((CHUNK,), jnp.int32),
            pltpu.VMEM((CHUNK, BD), jnp.float32),
            pltpu.SemaphoreType.DMA,
            pltpu.SemaphoreType.DMA,
        ),
    )
    def k(tA, iA, tB, iB, oA, oB, ia_v, ra_v, ib_v, rb_v, sA, sB):
        wid = lax.axis_index("s") * NC + lax.axis_index("c")

        def body(j, carry):
            c = wid + j * NW

            @pl.when(c < NCHUNK)
            def _():
                base = c * CHUNK
                pltpu.sync_copy(iA.at[pl.ds(base, CHUNK)], ia_v)
                pltpu.sync_copy(iB.at[pl.ds(base, CHUNK)], ib_v)
                cpA = pltpu.async_copy(tA.at[ia_v], ra_v, sA)
                cpB = pltpu.async_copy(tB.at[ib_v], rb_v, sB)
                cpA.wait()
                cpB.wait()
                pltpu.sync_copy(ra_v, oA.at[pl.ds(base, CHUNK)])
                pltpu.sync_copy(rb_v, oB.at[pl.ds(base, CHUNK)])

            return carry

        lax.fori_loop(0, niter, body, 0)

    return k(tabA, idxA, tabB, idxB)


def _scatter_pair(mL, idx_r, mR, idx_l, zeros_n):
    niter = (NCHUNK + NS - 1) // NS

    @functools.partial(
        pl.kernel,
        out_type=(
            jax.ShapeDtypeStruct((N, BD), jnp.float32),
            jax.ShapeDtypeStruct((N, BD), jnp.float32),
        ),
        mesh=_mesh,
        scratch_types=(
            pltpu.VMEM((CHUNK,), jnp.int32),
            pltpu.VMEM((CHUNK, BD), jnp.float32),
            pltpu.VMEM_SHARED((N, BD), jnp.float32),
        ),
    )
    def k(mLr, iR, mRr, iL, zr, oL, oR, idx_v, rows_v, acc):
        cid = lax.axis_index("c")
        sid = lax.axis_index("s")
        r0 = pl.multiple_of(sid * ROWS_A, 8)

        @pl.when(sid < NS - 1)
        def _():
            pltpu.sync_copy(zr.at[pl.ds(r0, ROWS_A)], acc.at[pl.ds(r0, ROWS_A)])

        @pl.when(sid == NS - 1)
        def _():
            pltpu.sync_copy(
                zr.at[pl.ds(r0, ROWS_LAST)], acc.at[pl.ds(r0, ROWS_LAST)]
            )

        plsc.subcore_barrier()

        def body(j, carry):
            c = sid + j * NS

            @pl.when(c < NCHUNK)
            def _():
                base = c * CHUNK

                @pl.when(cid == 0)
                def _():
                    pltpu.sync_copy(iR.at[pl.ds(base, CHUNK)], idx_v)
                    pltpu.sync_copy(mLr.at[pl.ds(base, CHUNK)], rows_v)
                    pltpu.sync_copy(rows_v, acc.at[idx_v], add=True)

                @pl.when(cid == 1)
                def _():
                    pltpu.sync_copy(iL.at[pl.ds(base, CHUNK)], idx_v)
                    pltpu.sync_copy(mRr.at[pl.ds(base, CHUNK)], rows_v)
                    pltpu.sync_copy(rows_v, acc.at[idx_v], add=True)

            return carry

        lax.fori_loop(0, niter, body, 0)
        plsc.subcore_barrier()

        @pl.when(jnp.logical_and(cid == 0, sid < NS - 1))
        def _():
            pltpu.sync_copy(acc.at[pl.ds(r0, ROWS_A)], oL.at[pl.ds(r0, ROWS_A)])

        @pl.when(jnp.logical_and(cid == 0, sid == NS - 1))
        def _():
            pltpu.sync_copy(
                acc.at[pl.ds(r0, ROWS_LAST)], oL.at[pl.ds(r0, ROWS_LAST)]
            )

        @pl.when(jnp.logical_and(cid == 1, sid < NS - 1))
        def _():
            pltpu.sync_copy(acc.at[pl.ds(r0, ROWS_A)], oR.at[pl.ds(r0, ROWS_A)])

        @pl.when(jnp.logical_and(cid == 1, sid == NS - 1))
        def _():
            pltpu.sync_copy(
                acc.at[pl.ds(r0, ROWS_LAST)], oR.at[pl.ds(r0, ROWS_LAST)]
            )

    return k(mL, idx_r, mR, idx_l, zeros_n)


TILE = 512
GRID = E // TILE


def _ffn_body(hb_ref, hl_ref, hr_ref, wbl, wnl, w1l, b1l, w2l, b2l, wbr, wnr,
              w1r, b1r, w2r, b2r, wnlt, wnrt, wst, bsk, ml_ref, mr_ref,
              sk_ref):
    hb = hb_ref[...]
    hl = hl_ref[...]
    hr = hr_ref[...]
    f32 = jnp.float32

    interL = jnp.dot(hb, wbl[...], preferred_element_type=f32) * jnp.dot(
        hl, wnl[...], preferred_element_type=f32)
    aL = jnp.maximum(
        jnp.dot(interL, w1l[...], preferred_element_type=f32) + b1l[...], 0.0)
    ml_ref[...] = jnp.dot(aL, w2l[...], preferred_element_type=f32) + b2l[...]

    interR = jnp.dot(hb, wbr[...], preferred_element_type=f32) * jnp.dot(
        hr, wnr[...], preferred_element_type=f32)
    aR = jnp.maximum(
        jnp.dot(interR, w1r[...], preferred_element_type=f32) + b1r[...], 0.0)
    mr_ref[...] = jnp.dot(aR, w2r[...], preferred_element_type=f32) + b2r[...]

    sk_ref[...] = (
        jnp.dot(hl, wnlt[...], preferred_element_type=f32)
        + jnp.dot(hr, wnrt[...], preferred_element_type=f32)
        + jnp.dot(hb, wst[...], preferred_element_type=f32)
        + bsk[...]
    )


def _final_body(ml_ref, mr_ref, sk_ref, ln_g, ln_b, wot, bo, out_ref):
    x = ml_ref[...] + mr_ref[...] + sk_ref[...]
    m = jnp.mean(x, axis=-1, keepdims=True)
    xc = x - m
    v = jnp.mean(xc * xc, axis=-1, keepdims=True)
    xn = xc * lax.rsqrt(v + 1e-5) * ln_g[...] + ln_b[...]
    out_ref[...] = (
        jnp.dot(jnp.maximum(xn, 0.0), wot[...],
                preferred_element_type=jnp.float32)
        + bo[...]
    )


def _edge_spec():
    return pl.BlockSpec((TILE, BD), lambda i: (i, 0))


def _w_spec(r, c):
    return pl.BlockSpec((r, c), lambda i: (0, 0))


def kernel(h_bond, bond_index, h_node, fL_Wb, fL_Wn, fL_W1, fL_b1, fL_W2,
           fL_b2, fR_Wb, fR_Wn, fR_W1, fR_b1, fR_W2, fR_b2, Wnl, bnl, Wnr,
           bnr, Ws, bs, ln_g, ln_b, Wo, bo):
    left = bond_index[0]
    right = bond_index[1]

    hnL, hnR = _gather_pair(h_node, left, h_node, right)

    bsk = (bnl + bnr + bs).reshape(1, BD)
    mL, mR, skip = pl.pallas_call(
        _ffn_body,
        grid=(GRID,),
        in_specs=[
            _edge_spec(), _edge_spec(), _edge_spec(),
            _w_spec(BD, ID), _w_spec(ND, ID), _w_spec(ID, ID), _w_spec(1, ID),
            _w_spec(ID, BD), _w_spec(1, BD),
            _w_spec(BD, ID), _w_spec(ND, ID), _w_spec(ID, ID), _w_spec(1, ID),
            _w_spec(ID, BD), _w_spec(1, BD),
            _w_spec(ND, BD), _w_spec(ND, BD), _w_spec(BD, BD), _w_spec(1, BD),
        ],
        out_specs=[_edge_spec(), _edge_spec(), _edge_spec()],
        out_shape=[jax.ShapeDtypeStruct((E, BD), jnp.float32)] * 3,
    )(
        h_bond, hnL, hnR,
        fL_Wb.T, fL_Wn.T, fL_W1.T, fL_b1.reshape(1, ID), fL_W2.T,
        fL_b2.reshape(1, BD),
        fR_Wb.T, fR_Wn.T, fR_W1.T, fR_b1.reshape(1, ID), fR_W2.T,
        fR_b2.reshape(1, BD),
        Wnl.T, Wnr.T, Ws.T, bsk,
    )

    zeros_n = jnp.zeros((N, BD), jnp.float32)
    sL, sR = _scatter_pair(mL, right, mR, left, zeros_n)

    mLg, mRg = _gather_pair(sL, left, sR, right)

    out = pl.pallas_call(
        _final_body,
        grid=(GRID,),
        in_specs=[
            _edge_spec(), _edge_spec(), _edge_spec(),
            _w_spec(1, BD), _w_spec(1, BD), _w_spec(BD, BD), _w_spec(1, BD),
        ],
        out_specs=_edge_spec(),
        out_shape=jax.ShapeDtypeStruct((E, BD), jnp.float32),
    )(mLg, mRg, skip, ln_g.reshape(1, BD), ln_b.reshape(1, BD), Wo.T,
      bo.reshape(1, BD))
    return out


# speedup vs baseline: 2.0572x; 1.0025x over previous
import functools

import jax
import jax.numpy as jnp
from jax import lax
from jax.experimental import pallas as pl
from jax.experimental.pallas import tpu as pltpu
from jax.experimental.pallas import tpu_sc as plsc

N = 10000
E = 320000
BD = 128
ND = 128
ID = 256

NC = 2
NS = 16
NW = NC * NS

CHUNK = 128
NCHUNK = E // CHUNK
ROWS_A = 624
ROWS_LAST = N - ROWS_A * (NS - 1)

_mesh = plsc.VectorSubcoreMesh(core_axis_name="c", subcore_axis_name="s")


def _gather_pair(tabA, idxA, tabB, idxB):
    niter = (NCHUNK + NW - 1) // NW

    @functools.partial(
        pl.kernel,
        out_type=(
            jax.ShapeDtypeStruct((E, BD), jnp.float32),
            jax.ShapeDtypeStruct((E, BD), jnp.float32),
        ),
        mesh=_mesh,
        scratch_types=(
            pltpu.VMEM((CHUNK,), jnp.int32),
            pltpu.VMEM((CHUNK, BD), jnp.float32),
            pltpu.VMEM((CHUNK,), jnp.int32),
            pltpu.VMEM((CHUNK, BD), jnp.float32),
            pltpu.SemaphoreType.DMA,
            pltpu.SemaphoreType.DMA,
        ),
    )
    def k(tA, iA, tB, iB, oA, oB, ia_v, ra_v, ib_v, rb_v, sA, sB):
        wid = lax.axis_index("s") * NC + lax.axis_index("c")

        def body(j, carry):
            c = wid + j * NW

            @pl.when(c < NCHUNK)
            def _():
                base = c * CHUNK
                pltpu.sync_copy(iA.at[pl.ds(base, CHUNK)], ia_v)
                pltpu.sync_copy(iB.at[pl.ds(base, CHUNK)], ib_v)
                cpA = pltpu.async_copy(tA.at[ia_v], ra_v, sA)
                cpB = pltpu.async_copy(tB.at[ib_v], rb_v, sB)
                cpA.wait()
                cpB.wait()
                pltpu.sync_copy(ra_v, oA.at[pl.ds(base, CHUNK)])
                pltpu.sync_copy(rb_v, oB.at[pl.ds(base, CHUNK)])

            return carry

        lax.fori_loop(0, niter, body, 0)

    return k(tabA, idxA, tabB, idxB)


def _scatter_pair(mL, idx_r, mR, idx_l, zeros_n):
    niter = (NCHUNK + NS - 1) // NS

    @functools.partial(
        pl.kernel,
        out_type=(
            jax.ShapeDtypeStruct((N, BD), jnp.float32),
            jax.ShapeDtypeStruct((N, BD), jnp.float32),
        ),
        mesh=_mesh,
        scratch_types=(
            pltpu.VMEM((CHUNK,), jnp.int32),
            pltpu.VMEM((CHUNK, BD), jnp.float32),
            pltpu.VMEM_SHARED((N, BD), jnp.float32),
        ),
    )
    def k(mLr, iR, mRr, iL, zr, oL, oR, idx_v, rows_v, acc):
        cid = lax.axis_index("c")
        sid = lax.axis_index("s")
        r0 = pl.multiple_of(sid * ROWS_A, 8)

        @pl.when(sid < NS - 1)
        def _():
            pltpu.sync_copy(zr.at[pl.ds(r0, ROWS_A)], acc.at[pl.ds(r0, ROWS_A)])

        @pl.when(sid == NS - 1)
        def _():
            pltpu.sync_copy(
                zr.at[pl.ds(r0, ROWS_LAST)], acc.at[pl.ds(r0, ROWS_LAST)]
            )

        plsc.subcore_barrier()

        def body(j, carry):
            c = sid + j * NS

            @pl.when(c < NCHUNK)
            def _():
                base = c * CHUNK

                @pl.when(cid == 0)
                def _():
                    pltpu.sync_copy(iR.at[pl.ds(base, CHUNK)], idx_v)
                    pltpu.sync_copy(mLr.at[pl.ds(base, CHUNK)], rows_v)
                    pltpu.sync_copy(rows_v, acc.at[idx_v], add=True)

                @pl.when(cid == 1)
                def _():
                    pltpu.sync_copy(iL.at[pl.ds(base, CHUNK)], idx_v)
                    pltpu.sync_copy(mRr.at[pl.ds(base, CHUNK)], rows_v)
                    pltpu.sync_copy(rows_v, acc.at[idx_v], add=True)

            return carry

        lax.fori_loop(0, niter, body, 0)
        plsc.subcore_barrier()

        @pl.when(jnp.logical_and(cid == 0, sid < NS - 1))
        def _():
            pltpu.sync_copy(acc.at[pl.ds(r0, ROWS_A)], oL.at[pl.ds(r0, ROWS_A)])

        @pl.when(jnp.logical_and(cid == 0, sid == NS - 1))
        def _():
            pltpu.sync_copy(
                acc.at[pl.ds(r0, ROWS_LAST)], oL.at[pl.ds(r0, ROWS_LAST)]
            )

        @pl.when(jnp.logical_and(cid == 1, sid < NS - 1))
        def _():
            pltpu.sync_copy(acc.at[pl.ds(r0, ROWS_A)], oR.at[pl.ds(r0, ROWS_A)])

        @pl.when(jnp.logical_and(cid == 1, sid == NS - 1))
        def _():
            pltpu.sync_copy(
                acc.at[pl.ds(r0, ROWS_LAST)], oR.at[pl.ds(r0, ROWS_LAST)]
            )

    return k(mL, idx_r, mR, idx_l, zeros_n)


TILE = 512
GRID = E // TILE


def _ffn_body(hb_ref, hl_ref, hr_ref, wbl, wnl, w1l, b1l, w2l, b2l, wbr, wnr,
              w1r, b1r, w2r, b2r, wnlt, wnrt, wst, bsk, ml_ref, mr_ref,
              sk_ref):
    f32 = jnp.float32
    bf = jnp.bfloat16
    hb = hb_ref[...].astype(bf)
    hl = hl_ref[...].astype(bf)
    hr = hr_ref[...].astype(bf)

    interL = (jnp.dot(hb, wbl[...], preferred_element_type=f32) * jnp.dot(
        hl, wnl[...], preferred_element_type=f32))
    aL = jnp.maximum(
        jnp.dot(interL.astype(bf), w1l[...], preferred_element_type=f32)
        + b1l[...], 0.0)
    ml_ref[...] = jnp.dot(
        aL.astype(bf), w2l[...], preferred_element_type=f32) + b2l[...]

    interR = (jnp.dot(hb, wbr[...], preferred_element_type=f32) * jnp.dot(
        hr, wnr[...], preferred_element_type=f32))
    aR = jnp.maximum(
        jnp.dot(interR.astype(bf), w1r[...], preferred_element_type=f32)
        + b1r[...], 0.0)
    mr_ref[...] = jnp.dot(
        aR.astype(bf), w2r[...], preferred_element_type=f32) + b2r[...]

    sk_ref[...] = (
        jnp.dot(hl, wnlt[...], preferred_element_type=f32)
        + jnp.dot(hr, wnrt[...], preferred_element_type=f32)
        + jnp.dot(hb, wst[...], preferred_element_type=f32)
        + bsk[...]
    )


def _final_body(ml_ref, mr_ref, sk_ref, ln_g, ln_b, wot, bo, out_ref):
    x = ml_ref[...] + mr_ref[...] + sk_ref[...]
    m = jnp.mean(x, axis=-1, keepdims=True)
    xc = x - m
    v = jnp.mean(xc * xc, axis=-1, keepdims=True)
    xn = xc * lax.rsqrt(v + 1e-5) * ln_g[...] + ln_b[...]
    out_ref[...] = (
        jnp.dot(jnp.maximum(xn, 0.0), wot[...],
                preferred_element_type=jnp.float32)
        + bo[...]
    )


def _edge_spec():
    return pl.BlockSpec((TILE, BD), lambda i: (i, 0))


def _w_spec(r, c):
    return pl.BlockSpec((r, c), lambda i: (0, 0))


def kernel(h_bond, bond_index, h_node, fL_Wb, fL_Wn, fL_W1, fL_b1, fL_W2,
           fL_b2, fR_Wb, fR_Wn, fR_W1, fR_b1, fR_W2, fR_b2, Wnl, bnl, Wnr,
           bnr, Ws, bs, ln_g, ln_b, Wo, bo):
    left = bond_index[0]
    right = bond_index[1]

    hnL, hnR = _gather_pair(h_node, left, h_node, right)

    bsk = (bnl + bnr + bs).reshape(1, BD)
    mL, mR, skip = pl.pallas_call(
        _ffn_body,
        grid=(GRID,),
        in_specs=[
            _edge_spec(), _edge_spec(), _edge_spec(),
            _w_spec(BD, ID), _w_spec(ND, ID), _w_spec(ID, ID), _w_spec(1, ID),
            _w_spec(ID, BD), _w_spec(1, BD),
            _w_spec(BD, ID), _w_spec(ND, ID), _w_spec(ID, ID), _w_spec(1, ID),
            _w_spec(ID, BD), _w_spec(1, BD),
            _w_spec(ND, BD), _w_spec(ND, BD), _w_spec(BD, BD), _w_spec(1, BD),
        ],
        out_specs=[_edge_spec(), _edge_spec(), _edge_spec()],
        out_shape=[jax.ShapeDtypeStruct((E, BD), jnp.float32)] * 3,
    )(
        h_bond, hnL, hnR,
        fL_Wb.T.astype(jnp.bfloat16), fL_Wn.T.astype(jnp.bfloat16),
        fL_W1.T.astype(jnp.bfloat16), fL_b1.reshape(1, ID),
        fL_W2.T.astype(jnp.bfloat16), fL_b2.reshape(1, BD),
        fR_Wb.T.astype(jnp.bfloat16), fR_Wn.T.astype(jnp.bfloat16),
        fR_W1.T.astype(jnp.bfloat16), fR_b1.reshape(1, ID),
        fR_W2.T.astype(jnp.bfloat16), fR_b2.reshape(1, BD),
        Wnl.T.astype(jnp.bfloat16), Wnr.T.astype(jnp.bfloat16),
        Ws.T.astype(jnp.bfloat16), bsk,
    )

    zeros_n = jnp.zeros((N, BD), jnp.float32)
    sL, sR = _scatter_pair(mL, right, mR, left, zeros_n)

    mLg, mRg = _gather_pair(sL, left, sR, right)

    out = pl.pallas_call(
        _final_body,
        grid=(GRID,),
        in_specs=[
            _edge_spec(), _edge_spec(), _edge_spec(),
            _w_spec(1, BD), _w_spec(1, BD), _w_spec(BD, BD), _w_spec(1, BD),
        ],
        out_specs=_edge_spec(),
        out_shape=jax.ShapeDtypeStruct((E, BD), jnp.float32),
    )(mLg, mRg, skip, ln_g.reshape(1, BD), ln_b.reshape(1, BD), Wo.T,
      bo.reshape(1, BD))
    return out
